# speedup vs baseline: 5.5826x; 5.5826x over previous
import functools

import jax
import jax.numpy as jnp
import numpy as np
from jax.experimental import pallas as pl
from jax.experimental.pallas import tpu as pltpu

N = 10000
E = 320000
IN_FEATS = 128
HID = 16
OUT_FEATS = 64
H1 = 8

_BLK = 400


def _dense_body(x_ref, w_ref, o_ref):
    o_ref[...] = jnp.dot(x_ref[...], w_ref[...],
                         preferred_element_type=jnp.float32)


def _dense(x, wbig):
    n, k = x.shape
    m = wbig.shape[1]
    return pl.pallas_call(
        _dense_body,
        grid=(n // _BLK,),
        in_specs=[
            pl.BlockSpec((_BLK, k), lambda i: (i, 0)),
            pl.BlockSpec((k, m), lambda i: (0, 0)),
        ],
        out_specs=pl.BlockSpec((_BLK, m), lambda i: (i, 0)),
        out_shape=jax.ShapeDtypeStruct((n, m), jnp.float32),
    )(x, wbig)


def _combine1_body(num_ref, den_ref, exp_ref, b_ref, w_ref, o_ref):
    denx = jnp.dot(den_ref[...], exp_ref[...],
                   preferred_element_type=jnp.float32)
    h = num_ref[...] / (denx + 1e-9) + b_ref[...]
    h = jnp.maximum(h, 0.0)
    o_ref[...] = jnp.dot(h, w_ref[...], preferred_element_type=jnp.float32)


def _combine1(num, den, expand, b1, w2big):
    n = num.shape[0]
    c = num.shape[1]
    hh = den.shape[1]
    m = w2big.shape[1]
    return pl.pallas_call(
        _combine1_body,
        grid=(n // _BLK,),
        in_specs=[
            pl.BlockSpec((_BLK, c), lambda i: (i, 0)),
            pl.BlockSpec((_BLK, hh), lambda i: (i, 0)),
            pl.BlockSpec((hh, c), lambda i: (0, 0)),
            pl.BlockSpec((1, c), lambda i: (0, 0)),
            pl.BlockSpec((c, m), lambda i: (0, 0)),
        ],
        out_specs=pl.BlockSpec((_BLK, m), lambda i: (i, 0)),
        out_shape=jax.ShapeDtypeStruct((n, m), jnp.float32),
    )(num, den, expand, b1, w2big)


def _combine2_body(num_ref, den_ref, b_ref, o_ref):
    o_ref[...] = num_ref[...] / (den_ref[...] + 1e-9) + b_ref[...]


def _combine2(num, den, b2):
    n, c = num.shape
    return pl.pallas_call(
        _combine2_body,
        grid=(n // _BLK,),
        in_specs=[
            pl.BlockSpec((_BLK, c), lambda i: (i, 0)),
            pl.BlockSpec((_BLK, 1), lambda i: (i, 0)),
            pl.BlockSpec((1, c), lambda i: (0, 0)),
        ],
        out_specs=pl.BlockSpec((_BLK, c), lambda i: (i, 0)),
        out_shape=jax.ShapeDtypeStruct((n, c), jnp.float32),
    )(num, den, b2)


def _edge_phase(feat, el, er, src, dst, heads, d):
    e = el[src] + er[dst]
    w = jnp.exp(jnp.maximum(e, 0.2 * e))
    den = jax.ops.segment_sum(w, dst, num_segments=N)
    wx = jnp.repeat(w, d, axis=1)
    num = jax.ops.segment_sum(feat[src] * wx, dst, num_segments=N)
    return num, den


def kernel(x, edge_index, W1, attn_l1, attn_r1, b1, W2, attn_l2, attn_r2, b2):
    src = edge_index[0]
    dst = edge_index[1]

    al1 = jax.scipy.linalg.block_diag(
        *[attn_l1[h][:, None] for h in range(H1)])
    ar1 = jax.scipy.linalg.block_diag(
        *[attn_r1[h][:, None] for h in range(H1)])
    wbig1 = jnp.concatenate([W1, W1 @ al1, W1 @ ar1], axis=1)

    al2 = attn_l2.reshape(OUT_FEATS, 1)
    ar2 = attn_r2.reshape(OUT_FEATS, 1)
    wbig2 = jnp.concatenate([W2, W2 @ al2, W2 @ ar2], axis=1)

    expand1 = jnp.repeat(jnp.eye(H1, dtype=jnp.float32), HID, axis=1)

    p1 = _dense(x, wbig1)
    feat1, el1, er1 = p1[:, :128], p1[:, 128:136], p1[:, 136:144]

    num1, den1 = _edge_phase(feat1, el1, er1, src, dst, H1, HID)

    p2 = _combine1(num1, den1, expand1, b1.reshape(1, -1), wbig2)
    feat2, el2, er2 = p2[:, :64], p2[:, 64:65], p2[:, 65:66]

    num2, den2 = _edge_phase(feat2, el2, er2, src, dst, 1, OUT_FEATS)

    return _combine2(num2, den2, b2.reshape(1, -1))


# speedup vs baseline: 44.4114x; 7.9553x over previous
import functools

import jax
import jax.numpy as jnp
import numpy as np
from jax import lax
from jax.experimental import pallas as pl
from jax.experimental.pallas import tpu as pltpu
from jax.experimental.pallas import tpu_sc as plsc

N = 10000
E = 320000
IN_FEATS = 128
HID = 16
OUT_FEATS = 64
H1 = 8

_BLK = 400


def _dense_body(x_ref, w_ref, o_ref):
    o_ref[...] = jnp.dot(x_ref[...], w_ref[...],
                         preferred_element_type=jnp.float32)


def _dense(x, wbig):
    n, k = x.shape
    m = wbig.shape[1]
    return pl.pallas_call(
        _dense_body,
        grid=(n // _BLK,),
        in_specs=[
            pl.BlockSpec((_BLK, k), lambda i: (i, 0)),
            pl.BlockSpec((k, m), lambda i: (0, 0)),
        ],
        out_specs=pl.BlockSpec((_BLK, m), lambda i: (i, 0)),
        out_shape=jax.ShapeDtypeStruct((n, m), jnp.float32),
    )(x, wbig)


def _combine1_body(numa_ref, numb_ref, dena_ref, denb_ref, exp_ref, b_ref,
                   w_ref, o_ref):
    den = dena_ref[...] + denb_ref[...]
    denx = jnp.dot(den, exp_ref[...], preferred_element_type=jnp.float32)
    h = (numa_ref[...] + numb_ref[...]) / (denx + 1e-9) + b_ref[...]
    h = jnp.maximum(h, 0.0)
    o_ref[...] = jnp.dot(h, w_ref[...], preferred_element_type=jnp.float32)


def _combine1(numa, numb, dena, denb, expand, b1, w2big):
    n = numa.shape[0]
    c = numa.shape[1]
    hh = dena.shape[1]
    m = w2big.shape[1]
    blk = pl.BlockSpec((_BLK, c), lambda i: (i, 0))
    blkh = pl.BlockSpec((_BLK, hh), lambda i: (i, 0))
    return pl.pallas_call(
        _combine1_body,
        grid=(n // _BLK,),
        in_specs=[
            blk, blk, blkh, blkh,
            pl.BlockSpec((hh, c), lambda i: (0, 0)),
            pl.BlockSpec((1, c), lambda i: (0, 0)),
            pl.BlockSpec((c, m), lambda i: (0, 0)),
        ],
        out_specs=pl.BlockSpec((_BLK, m), lambda i: (i, 0)),
        out_shape=jax.ShapeDtypeStruct((n, m), jnp.float32),
    )(numa, numb, dena, denb, expand, b1, w2big)


def _combine2_body(numa_ref, numb_ref, dena_ref, denb_ref, b_ref, o_ref):
    den = dena_ref[...] + denb_ref[...]
    o_ref[...] = ((numa_ref[...] + numb_ref[...]) / (den[:, 0:1] + 1e-9)
                  + b_ref[...])


def _combine2(numa, numb, dena, denb, b2):
    n, c = numa.shape
    hh = dena.shape[1]
    blk = pl.BlockSpec((_BLK, c), lambda i: (i, 0))
    blkh = pl.BlockSpec((_BLK, hh), lambda i: (i, 0))
    return pl.pallas_call(
        _combine2_body,
        grid=(n // _BLK,),
        in_specs=[
            blk, blk, blkh, blkh,
            pl.BlockSpec((1, c), lambda i: (0, 0)),
        ],
        out_specs=pl.BlockSpec((_BLK, c), lambda i: (i, 0)),
        out_shape=jax.ShapeDtypeStruct((n, c), jnp.float32),
    )(numa, numb, dena, denb, b2)


_B = 128
_NSUB = 16
_HP = 16


@functools.lru_cache(maxsize=None)
def _make_edge_sc(C, H):
    D = C // H
    NB = E // _B
    NBC = NB // 2
    ROWS_T = (N // _NSUB) // 8 * 8
    ROWS_REM = N - ROWS_T * _NSUB

    mesh = plsc.VectorSubcoreMesh(core_axis_name="c", subcore_axis_name="s")

    @functools.partial(
        pl.kernel,
        out_type=(
            jax.ShapeDtypeStruct((2, N, C), jnp.float32),
            jax.ShapeDtypeStruct((2, N, _HP), jnp.float32),
        ),
        mesh=mesh,
        compiler_params=pltpu.CompilerParams(use_tc_tiling_on_sc=False),
        scratch_types=[
            pltpu.MemorySpace.VMEM_SHARED((N, C), jnp.float32),
            pltpu.MemorySpace.VMEM_SHARED((N, _HP), jnp.float32),
            pltpu.MemorySpace.VMEM((_B,), jnp.int32),
            pltpu.MemorySpace.VMEM((1, _B), jnp.int32),
            pltpu.MemorySpace.VMEM((_B, _HP), jnp.float32),
            pltpu.MemorySpace.VMEM((_B, _HP), jnp.float32),
            pltpu.MemorySpace.VMEM((_B, _HP), jnp.float32),
            pltpu.MemorySpace.VMEM((_B, C), jnp.float32),
            pltpu.SemaphoreType.DMA,
        ],
    )
    def edge_kernel(feat, el, er, src, dst, num_out, den_out,
                    num_acc, den_acc, sidx, didx, elb, erb, wb, fb, sem):
        c = lax.axis_index("c")
        s = lax.axis_index("s")
        iota = lax.iota(jnp.int32, 16)
        zeros16 = jnp.zeros((16,), jnp.float32)

        @pl.loop(0, _B * C // 16)
        def _zf(k):
            i = k // (C // 16)
            j = k % (C // 16)
            fb[i, pl.ds(j * 16, 16)] = zeros16

        @pl.loop(0, _B)
        def _zw(k):
            wb[k, :] = zeros16

        r0 = s * ROWS_T
        for q in range(4):
            pltpu.sync_copy(fb, num_acc.at[pl.ds(r0 + q * _B, _B)])
            pltpu.sync_copy(wb, den_acc.at[pl.ds(r0 + q * _B, _B)])
        rem = ROWS_T - 4 * _B
        pltpu.sync_copy(fb.at[pl.ds(0, rem)],
                        num_acc.at[pl.ds(r0 + 4 * _B, rem)])
        pltpu.sync_copy(wb.at[pl.ds(0, rem)],
                        den_acc.at[pl.ds(r0 + 4 * _B, rem)])

        @pl.when(s == _NSUB - 1)
        def _ztail():
            base = ROWS_T * _NSUB
            pltpu.sync_copy(fb.at[pl.ds(0, ROWS_REM)],
                            num_acc.at[pl.ds(base, ROWS_REM)])
            pltpu.sync_copy(wb.at[pl.ds(0, ROWS_REM)],
                            den_acc.at[pl.ds(base, ROWS_REM)])

        plsc.subcore_barrier()

        nmine = (NBC - s + _NSUB - 1) // _NSUB

        @pl.loop(0, nmine)
        def _blk(j):
            blk = c * NBC + s + j * _NSUB
            off = blk * _B
            pltpu.sync_copy(src.at[pl.ds(off, _B)], sidx)
            pltpu.sync_copy(dst.at[pl.ds(off, _B)], didx.at[0])
            pltpu.async_copy(el.at[sidx], elb, sem).wait()
            pltpu.async_copy(er.at[didx.at[0]], erb, sem).wait()
            pltpu.async_copy(feat.at[sidx], fb, sem).wait()

            @pl.loop(0, _B)
            def _w(k):
                ev = elb[k, :] + erb[k, :]
                wb[k, :] = jnp.exp(jnp.maximum(ev, 0.2 * ev))

            @pl.loop(0, _B)
            def _m(i):
                wrow = wb[i, :]
                for h in range(H):
                    wv = jnp.broadcast_to(wrow[h], (16,))
                    for t in range(D // 16):
                        col = h * D + t * 16
                        fb[i, pl.ds(col, 16)] = fb[i, pl.ds(col, 16)] * wv

            pltpu.sync_copy(fb, num_acc.at[didx.at[0]], add=True)
            pltpu.sync_copy(wb, den_acc.at[didx.at[0]], add=True)

        plsc.subcore_barrier()
        pltpu.sync_copy(num_acc.at[pl.ds(r0, ROWS_T)],
                        num_out.at[c, pl.ds(r0, ROWS_T)])
        pltpu.sync_copy(den_acc.at[pl.ds(r0, ROWS_T)],
                        den_out.at[c, pl.ds(r0, ROWS_T)])

        @pl.when(s == _NSUB - 1)
        def _ftail():
            base = ROWS_T * _NSUB
            pltpu.sync_copy(num_acc.at[pl.ds(base, ROWS_REM)],
                            num_out.at[c, pl.ds(base, ROWS_REM)])
            pltpu.sync_copy(den_acc.at[pl.ds(base, ROWS_REM)],
                            den_out.at[c, pl.ds(base, ROWS_REM)])

    return edge_kernel


def kernel(x, edge_index, W1, attn_l1, attn_r1, b1, W2, attn_l2, attn_r2, b2):
    src = edge_index[0]
    dst = edge_index[1]

    al1 = jax.scipy.linalg.block_diag(
        *[attn_l1[h][:, None] for h in range(H1)])
    ar1 = jax.scipy.linalg.block_diag(
        *[attn_r1[h][:, None] for h in range(H1)])
    zp1 = jnp.zeros((IN_FEATS, _HP - H1), jnp.float32)
    wbig1 = jnp.concatenate([W1, W1 @ al1, zp1, W1 @ ar1, zp1],
                            axis=1)

    al2 = attn_l2.reshape(OUT_FEATS, 1)
    ar2 = attn_r2.reshape(OUT_FEATS, 1)
    zpad = jnp.zeros((H1 * HID, _HP - 1), jnp.float32)
    wbig2 = jnp.concatenate([W2, W2 @ al2, zpad, W2 @ ar2, zpad],
                            axis=1)

    expand1 = jnp.concatenate(
        [jnp.repeat(jnp.eye(H1, dtype=jnp.float32), HID, axis=1),
         jnp.zeros((_HP - H1, H1 * HID), jnp.float32)], axis=0)

    p1 = _dense(x, wbig1)
    feat1 = p1[:, :128]
    el1 = p1[:, 128:144]
    er1 = p1[:, 144:160]

    num1p, den1p = _make_edge_sc(IN_FEATS, H1)(feat1, el1, er1, src, dst)

    p2 = _combine1(num1p[0], num1p[1], den1p[0], den1p[1],
                   expand1, b1.reshape(1, -1), wbig2)
    feat2 = p2[:, :64]
    el2 = p2[:, 64:80]
    er2 = p2[:, 80:96]

    num2p, den2p = _make_edge_sc(OUT_FEATS, 1)(feat2, el2, er2, src, dst)

    return _combine2(num2p[0], num2p[1], den2p[0], den2p[1],
                     b2.reshape(1, -1))


# speedup vs baseline: 60.4640x; 1.3615x over previous
import functools

import jax
import jax.numpy as jnp
from jax import lax
from jax.experimental import pallas as pl
from jax.experimental.pallas import tpu as pltpu
from jax.experimental.pallas import tpu_sc as plsc

N = 10000
E = 320000
IN_FEATS = 128
HID = 16
OUT_FEATS = 64
H1 = 8

_BLK = 400

_B = 64
_NSUB = 16
_HP = 16


def _dense_body(x_ref, w_ref, o1_ref, o2_ref):
    z = jnp.dot(x_ref[...], w_ref[...], preferred_element_type=jnp.float32)
    cfel = o1_ref.shape[1]
    o1_ref[...] = z[:, :cfel]
    o2_ref[...] = z[:, cfel:]


def _dense(x, wbig, cfel):
    n, k = x.shape
    m = wbig.shape[1]
    return pl.pallas_call(
        _dense_body,
        grid=(n // _BLK,),
        in_specs=[
            pl.BlockSpec((_BLK, k), lambda i: (i, 0)),
            pl.BlockSpec((k, m), lambda i: (0, 0)),
        ],
        out_specs=[
            pl.BlockSpec((_BLK, cfel), lambda i: (i, 0)),
            pl.BlockSpec((_BLK, m - cfel), lambda i: (i, 0)),
        ],
        out_shape=[
            jax.ShapeDtypeStruct((n, cfel), jnp.float32),
            jax.ShapeDtypeStruct((n, m - cfel), jnp.float32),
        ],
    )(x, wbig)


def _combine1_body(pa_ref, pb_ref, exp_ref, b_ref, w_ref, o1_ref, o2_ref):
    pa = pa_ref[...] + pb_ref[...]
    num = pa[:, :IN_FEATS]
    den = pa[:, IN_FEATS:]
    denx = jnp.dot(den, exp_ref[...], preferred_element_type=jnp.float32)
    h = num / (denx + 1e-9) + b_ref[...]
    h = jnp.maximum(h, 0.0)
    z = jnp.dot(h, w_ref[...], preferred_element_type=jnp.float32)
    cfel = o1_ref.shape[1]
    o1_ref[...] = z[:, :cfel]
    o2_ref[...] = z[:, cfel:]


def _combine1(pa, pb, expand, b1, w2big, cfel):
    n = pa.shape[0]
    cc = pa.shape[1]
    m = w2big.shape[1]
    blk = pl.BlockSpec((_BLK, cc), lambda i: (i, 0))
    return pl.pallas_call(
        _combine1_body,
        grid=(n // _BLK,),
        in_specs=[
            blk, blk,
            pl.BlockSpec((_HP, IN_FEATS), lambda i: (0, 0)),
            pl.BlockSpec((1, IN_FEATS), lambda i: (0, 0)),
            pl.BlockSpec((IN_FEATS, m), lambda i: (0, 0)),
        ],
        out_specs=[
            pl.BlockSpec((_BLK, cfel), lambda i: (i, 0)),
            pl.BlockSpec((_BLK, m - cfel), lambda i: (i, 0)),
        ],
        out_shape=[
            jax.ShapeDtypeStruct((n, cfel), jnp.float32),
            jax.ShapeDtypeStruct((n, m - cfel), jnp.float32),
        ],
    )(pa, pb, expand, b1, w2big)


def _combine2_body(pa_ref, pb_ref, b_ref, o_ref):
    pa = pa_ref[...] + pb_ref[...]
    num = pa[:, :OUT_FEATS]
    den = pa[:, OUT_FEATS:OUT_FEATS + 1]
    o_ref[...] = num / (den + 1e-9) + b_ref[...]


def _combine2(pa, pb, b2):
    n = pa.shape[0]
    cc = pa.shape[1]
    blk = pl.BlockSpec((_BLK, cc), lambda i: (i, 0))
    return pl.pallas_call(
        _combine2_body,
        grid=(n // _BLK,),
        in_specs=[
            blk, blk,
            pl.BlockSpec((1, OUT_FEATS), lambda i: (0, 0)),
        ],
        out_specs=pl.BlockSpec((_BLK, OUT_FEATS), lambda i: (i, 0)),
        out_shape=jax.ShapeDtypeStruct((n, OUT_FEATS), jnp.float32),
    )(pa, pb, b2)


@functools.lru_cache(maxsize=None)
def _make_edge_sc(C, H):
    D = C // H
    CC = C + _HP
    NB = E // _B
    NBC = NB // 2
    NMAX = (NBC + _NSUB - 1) // _NSUB
    ROWS_T = (N // _NSUB) // 8 * 8
    ROWS_REM = N - ROWS_T * _NSUB

    mesh = plsc.VectorSubcoreMesh(core_axis_name="c", subcore_axis_name="s")

    def buf_pair(shape, dtype):
        return [pltpu.MemorySpace.VMEM(shape, dtype) for _ in range(2)]

    @functools.partial(
        pl.kernel,
        out_type=jax.ShapeDtypeStruct((2, N, CC), jnp.float32),
        mesh=mesh,
        compiler_params=pltpu.CompilerParams(use_tc_tiling_on_sc=False),
        scratch_types=[
            pltpu.MemorySpace.VMEM_SHARED((N, CC), jnp.float32),
            buf_pair((_B,), jnp.int32),
            buf_pair((1, _B), jnp.int32),
            buf_pair((1, _B), jnp.int32),
            buf_pair((_B, _HP), jnp.float32),
            buf_pair((_B, CC), jnp.float32),
            buf_pair((_B, CC), jnp.float32),
            [pltpu.SemaphoreType.DMA for _ in range(2)],
            [pltpu.SemaphoreType.DMA for _ in range(2)],
        ],
    )
    def edge_kernel(fel, er, src, dst, acc_out,
                    acc, sidx, didx, didxs, erb, fbg, mb, gsem, ssem):
        c = lax.axis_index("c")
        s = lax.axis_index("s")
        zeros16 = jnp.zeros((16,), jnp.float32)

        nmine = (NBC - s + _NSUB - 1) // _NSUB

        def issue_gathers(b, k):
            off = (c * NBC + s + k * _NSUB) * _B
            pltpu.sync_copy(src.at[pl.ds(off, _B)], sidx[b])
            pltpu.sync_copy(dst.at[pl.ds(off, _B)], didx[b].at[0])
            pltpu.async_copy(fel.at[sidx[b]], fbg[b], gsem[b])
            pltpu.async_copy(er.at[didx[b].at[0]], erb[b], gsem[b])

        def wait_gathers(b):
            pltpu.make_async_copy(fel.at[sidx[b]], fbg[b], gsem[b]).wait()
            pltpu.make_async_copy(er.at[didx[b].at[0]], erb[b], gsem[b]).wait()

        def issue_scatter(b):
            pltpu.async_copy(mb[b], acc.at[didxs[b].at[0]], ssem[b], add=True)

        def wait_scatter(b):
            pltpu.make_async_copy(mb[b], acc.at[didxs[b].at[0]],
                                  ssem[b]).wait()

        def compute(b):
            @pl.loop(0, _B // 16)
            def _ci(k):
                didxs[b][0, pl.ds(k * 16, 16)] = didx[b][0, pl.ds(k * 16, 16)]

            @pl.loop(0, _B)
            def _m(i):
                ev = fbg[b][i, pl.ds(C, 16)] + erb[b][i, :]
                wrow = jnp.exp(jnp.maximum(ev, 0.2 * ev))
                mb[b][i, pl.ds(C, 16)] = wrow
                for h in range(H):
                    wv = jnp.broadcast_to(wrow[h], (16,))
                    for t in range(D // 16):
                        col = h * D + t * 16
                        mb[b][i, pl.ds(col, 16)] = fbg[b][i, pl.ds(col, 16)] * wv

        for b in range(2):
            @pl.when(b < nmine)
            def _prime():
                issue_gathers(b, b)

        @pl.loop(0, _B * CC // 16)
        def _zf(k):
            i = k // (CC // 16)
            j = k % (CC // 16)
            mb[0][i, pl.ds(j * 16, 16)] = zeros16

        r0 = s * ROWS_T
        nfull = ROWS_T // _B
        rem = ROWS_T - nfull * _B
        for q in range(nfull):
            pltpu.sync_copy(mb[0], acc.at[pl.ds(r0 + q * _B, _B)])
        if rem:
            pltpu.sync_copy(mb[0].at[pl.ds(0, rem)],
                            acc.at[pl.ds(r0 + nfull * _B, rem)])

        @pl.when(s == _NSUB - 1)
        def _ztail():
            base = ROWS_T * _NSUB
            pltpu.sync_copy(mb[0].at[pl.ds(0, ROWS_REM)],
                            acc.at[pl.ds(base, ROWS_REM)])

        plsc.subcore_barrier()

        @pl.loop(0, (NMAX + 1) // 2)
        def _blk(p):
            for b in range(2):
                k = p * 2 + b

                @pl.when(k < nmine)
                def _do():
                    wait_gathers(b)

                    @pl.when(k >= 2)
                    def _ws():
                        wait_scatter(b)

                    compute(b)
                    issue_scatter(b)

                    @pl.when(k + 2 < nmine)
                    def _nx():
                        issue_gathers(b, k + 2)

        for b in range(2):
            @pl.when(b < nmine)
            def _drain():
                wait_scatter(b)

        plsc.subcore_barrier()
        pltpu.sync_copy(acc.at[pl.ds(r0, ROWS_T)],
                        acc_out.at[c, pl.ds(r0, ROWS_T)])

        @pl.when(s == _NSUB - 1)
        def _ftail():
            base = ROWS_T * _NSUB
            pltpu.sync_copy(acc.at[pl.ds(base, ROWS_REM)],
                            acc_out.at[c, pl.ds(base, ROWS_REM)])

    return edge_kernel


def kernel(x, edge_index, W1, attn_l1, attn_r1, b1, W2, attn_l2, attn_r2, b2):
    src = edge_index[0]
    dst = edge_index[1]

    al1 = jax.scipy.linalg.block_diag(
        *[attn_l1[h][:, None] for h in range(H1)])
    ar1 = jax.scipy.linalg.block_diag(
        *[attn_r1[h][:, None] for h in range(H1)])
    zp1 = jnp.zeros((IN_FEATS, _HP - H1), jnp.float32)
    wbig1 = jnp.concatenate([W1, W1 @ al1, zp1, W1 @ ar1, zp1],
                            axis=1)

    al2 = attn_l2.reshape(OUT_FEATS, 1)
    ar2 = attn_r2.reshape(OUT_FEATS, 1)
    zp2 = jnp.zeros((H1 * HID, _HP - 1), jnp.float32)
    wbig2 = jnp.concatenate([W2, W2 @ al2, zp2, W2 @ ar2, zp2],
                            axis=1)

    expand1 = jnp.concatenate(
        [jnp.repeat(jnp.eye(H1, dtype=jnp.float32), HID, axis=1),
         jnp.zeros((_HP - H1, H1 * HID), jnp.float32)], axis=0)

    fel1, er1 = _dense(x, wbig1, IN_FEATS + _HP)

    p1 = _make_edge_sc(IN_FEATS, H1)(fel1, er1, src, dst)

    fel2, er2 = _combine1(p1[0], p1[1], expand1, b1.reshape(1, -1), wbig2,
                          OUT_FEATS + _HP)

    p2 = _make_edge_sc(OUT_FEATS, 1)(fel2, er2, src, dst)

    return _combine2(p2[0], p2[1], b2.reshape(1, -1))
